# Initial kernel scaffold; baseline (speedup 1.0000x reference)
import jax
import jax.numpy as jnp
from jax.experimental import pallas as pl


def kernel(node, edge_index, edge_attr, batch_ptr, Wrel, brel, Wroot, W1, b1, W2, b2, g1, be1, g2, be2, Wjk, bjk):
    raise NotImplementedError("write your pallas kernel here")


# speedup vs baseline: 3.0800x; 3.0800x over previous
import functools

import jax
import jax.numpy as jnp
from jax import lax
from jax.experimental import pallas as pl
from jax.experimental.pallas import tpu as pltpu
from jax.experimental.pallas import tpu_sc as plsc

NC = 2
NS = 16
NW = NC * NS
K = 128


def _make_sc_agg(Npad, D, NB, with_deg):
    mesh = plsc.VectorSubcoreMesh(core_axis_name="c", subcore_axis_name="s")
    RPT = Npad // NS
    NQ = 4
    CH = RPT // NQ

    out_type = [jax.ShapeDtypeStruct((NC, Npad, D), jnp.float32)]
    scratch = [
        pltpu.VMEM((K,), jnp.int32),
        pltpu.VMEM((K,), jnp.int32),
        pltpu.VMEM((K,), jnp.float32),
        pltpu.VMEM((K, D), jnp.float32),
        pltpu.VMEM((CH, D), jnp.float32),
        pltpu.VMEM_SHARED((Npad, D), jnp.float32),
        pltpu.SemaphoreType.DMA,
    ]
    if with_deg:
        out_type.append(jax.ShapeDtypeStruct((NC * Npad,), jnp.float32))
        scratch.append(pltpu.VMEM((K,), jnp.float32))
        scratch.append(pltpu.VMEM((RPT,), jnp.float32))
        scratch.append(pltpu.VMEM_SHARED((Npad,), jnp.float32))

    @functools.partial(
        pl.kernel, mesh=mesh, out_type=out_type, scratch_types=scratch,
        name="sc_agg_deg" if with_deg else "sc_agg",
    )
    def k(*refs):
        if with_deg:
            (x_hbm, srcs, dsts, eas, msk, z2, z1,
             out_hbm, outd_hbm,
             src_v, dst_v, ea_v, rows_v, stg, agg_sh, sem,
             m_v, stg1, deg_sh) = refs
        else:
            (x_hbm, srcs, dsts, eas, z2,
             out_hbm,
             src_v, dst_v, ea_v, rows_v, stg, agg_sh, sem) = refs
        c = lax.axis_index("c")
        s = lax.axis_index("s")
        wid = s * NC + c
        r0 = s * RPT
        pltpu.sync_copy(z2, stg)
        for q in range(NQ):
            pltpu.sync_copy(stg, agg_sh.at[pl.ds(r0 + q * CH, CH)])
        if with_deg:
            pltpu.sync_copy(z1, stg1)
            pltpu.sync_copy(stg1, deg_sh.at[pl.ds(r0, RPT)])
        plsc.subcore_barrier()

        base0 = wid * (NB * K)

        def blk(b, carry):
            base = base0 + b * K
            pltpu.sync_copy(srcs.at[pl.ds(base, K)], src_v)
            pltpu.sync_copy(dsts.at[pl.ds(base, K)], dst_v)
            pltpu.sync_copy(eas.at[pl.ds(base, K)], ea_v)
            pltpu.async_copy(x_hbm.at[src_v], rows_v, sem).wait()

            def srow16(jj, cc):
                ea16 = ea_v[pl.ds(jj * 16, 16)]
                for l in range(16):
                    scl = ea16[l]
                    r = jj * 16 + l
                    for d in range(D // 16):
                        sl = pl.ds(d * 16, 16)
                        rows_v[r, sl] = rows_v[r, sl] * scl
                return cc

            lax.fori_loop(0, K // 16, srow16, 0)
            pltpu.sync_copy(rows_v, agg_sh.at[dst_v], add=True)
            if with_deg:
                pltpu.sync_copy(msk.at[pl.ds(base, K)], m_v)
                pltpu.sync_copy(m_v, deg_sh.at[dst_v], add=True)
            return carry

        lax.fori_loop(0, NB, blk, 0)
        plsc.subcore_barrier()
        for q in range(NQ):
            pltpu.sync_copy(agg_sh.at[pl.ds(r0 + q * CH, CH)], stg)
            pltpu.sync_copy(stg, out_hbm.at[c, pl.ds(r0 + q * CH, CH)])
        if with_deg:
            pltpu.sync_copy(deg_sh.at[pl.ds(r0, RPT)], stg1)
            pltpu.sync_copy(stg1, outd_hbm.at[pl.ds(c * Npad + r0, RPT)])

    return k


def _deginv_body(degp_ref, out_ref):
    d0 = degp_ref[0:1, :]
    d1 = degp_ref[1:2, :]
    out_ref[...] = 1.0 / jnp.maximum(d0 + d1, 1.0)


def _deginv(degp):
    _, Npad = degp.shape
    return pl.pallas_call(
        _deginv_body,
        out_shape=jax.ShapeDtypeStruct((1, Npad), jnp.float32),
    )(degp)


def _ln(h, g, b, eps=1e-5):
    mu = jnp.mean(h, axis=-1, keepdims=True)
    var = jnp.mean((h - mu) ** 2, axis=-1, keepdims=True)
    return (h - mu) / jnp.sqrt(var + eps) * g + b


def _dense_body(a0, a1, dinv, x, acc, WrelT, brel, WrootT, W1T, b1,
                W2T, b2, g1, be1, g2, be2, WjkT, xo, acco):
    f32 = jnp.float32
    agg = (a0[...] + a1[...]) * dinv[...]
    xb = x[...]
    x1 = (jnp.dot(agg, WrelT[...], preferred_element_type=f32)
          + jnp.dot(xb, WrootT[...], preferred_element_type=f32)
          + brel[...])
    h = jnp.dot(x1, W1T[...], preferred_element_type=f32) + b1[...]
    x2 = jnp.maximum(_ln(h, g1[...], be1[...]), 0.0)
    h2 = jnp.dot(x2, W2T[...], preferred_element_type=f32) + b2[...]
    x3 = jnp.maximum(_ln(h2, g2[...], be2[...]), 0.0)
    xo[...] = x3
    acco[...] = acc[...] + jnp.dot(x3, WjkT[...], preferred_element_type=f32)


def _dense_layer(a0, a1, dinv, x, acc, WrelT, brel, WrootT, W1T, b1,
                 W2T, b2, g1, be1, g2, be2, WjkT):
    Npad, D = x.shape
    R = 256
    grid = (Npad // R,)
    rb = pl.BlockSpec((R, D), lambda i: (i, 0))
    db = pl.BlockSpec((R, 1), lambda i: (i, 0))
    wb = pl.BlockSpec((D, D), lambda i: (0, 0))
    bb = pl.BlockSpec((1, D), lambda i: (0, 0))
    return pl.pallas_call(
        _dense_body,
        grid=grid,
        in_specs=[rb, rb, db, rb, rb, wb, bb, wb, wb, bb,
                  wb, bb, bb, bb, bb, bb, wb],
        out_specs=[rb, rb],
        out_shape=[jax.ShapeDtypeStruct((Npad, D), jnp.float32),
                   jax.ShapeDtypeStruct((Npad, D), jnp.float32)],
    )(a0, a1, dinv, x, acc, WrelT, brel, WrootT, W1T, b1,
      W2T, b2, g1, be1, g2, be2, WjkT)


def kernel(node, edge_index, edge_attr, batch_ptr, Wrel, brel, Wroot,
           W1, b1, W2, b2, g1, be1, g2, be2, Wjk, bjk):
    N, D = node.shape
    E = edge_index.shape[1]
    L = Wrel.shape[0]
    f32 = jnp.float32

    Npad = ((N + 255) // 256) * 256
    NB = -(-E // (NW * K))
    Epad = NW * K * NB

    src = jnp.pad(edge_index[0], (0, Epad - E))
    dst = jnp.pad(edge_index[1], (0, Epad - E))
    ea = jnp.pad(edge_attr, (0, Epad - E))
    emask = jnp.pad(jnp.ones((E,), f32), (0, Epad - E))
    z2 = jnp.zeros((Npad // NS // 4, D), f32)
    z1 = jnp.zeros((Npad // NS,), f32)

    x = jnp.pad(node, ((0, Npad - N), (0, 0)))
    acc = jnp.broadcast_to(bjk[None, :], (Npad, D))

    sc_deg = _make_sc_agg(Npad, D, NB, True)
    sc = _make_sc_agg(Npad, D, NB, False)

    dinv = None
    for i in range(L):
        if i == 0:
            aggp, degp = sc_deg(x, src, dst, ea, emask, z2, z1)
            dinv = jnp.reshape(_deginv(jnp.reshape(degp, (NC, Npad))),
                               (Npad, 1))
        else:
            (aggp,) = sc(x, src, dst, ea, z2)
        x, acc = _dense_layer(
            aggp[0], aggp[1], dinv, x, acc,
            Wrel[i].T, brel[i][None, :], Wroot[i].T,
            W1[i].T, b1[i][None, :], W2[i].T, b2[i][None, :],
            g1[i][None, :], be1[i][None, :], g2[i][None, :], be2[i][None, :],
            Wjk[:, i * D:(i + 1) * D].T)
    return acc[:N]
